# baseline (speedup 1.0000x reference)
import functools

import jax
import jax.numpy as jnp
from jax import lax
from jax.experimental import pallas as pl
from jax.experimental.pallas import tpu as pltpu
from jax.experimental.pallas import tpu_sc as plsc

_EPS = 1e-4
_B = 16
_K = 500
_KP = 512
_HW = 128 * 128
_CWH = 10
_CREG = 2


def _sc_gather_body(wh_hbm, reg_hbm, ct_hbm, idx_hbm,
                    out_wh, out_reg, out_ct,
                    plane_v, idx_v, g_v):
    cid = lax.axis_index("c")
    sid = lax.axis_index("s")
    wid = sid * 2 + cid
    b = wid // 2
    h = wid % 2

    pltpu.sync_copy(idx_hbm.at[b], idx_v)

    def do_plane(src_hbm, src_row, dst_hbm, dst_row):
        pltpu.sync_copy(src_hbm.at[src_row], plane_v)
        for j in range(_KP // 16):
            iv = idx_v[pl.ds(j * 16, 16)]
            g_v[pl.ds(j * 16, 16)] = plsc.load_gather(plane_v, [iv])
        pltpu.sync_copy(g_v, dst_hbm.at[dst_row])

    for j in range(5):
        c = h * 5 + j
        do_plane(wh_hbm, b * _CWH + c, out_wh, b * _CWH + c)
    do_plane(reg_hbm, b * _CREG + h, out_reg, b * _CREG + h)

    @pl.when(h == 0)
    def _():
        do_plane(ct_hbm, b, out_ct, b)


_sc_gather = functools.partial(
    pl.kernel,
    mesh=plsc.VectorSubcoreMesh(core_axis_name="c", subcore_axis_name="s"),
    out_type=(
        jax.ShapeDtypeStruct((_B * _CWH, _KP), jnp.float32),
        jax.ShapeDtypeStruct((_B * _CREG, _KP), jnp.float32),
        jax.ShapeDtypeStruct((_B, _KP), jnp.float32),
    ),
    scratch_types=[
        pltpu.VMEM((_HW,), jnp.float32),
        pltpu.VMEM((_KP,), jnp.int32),
        pltpu.VMEM((_KP,), jnp.float32),
    ],
    compiler_params=pltpu.CompilerParams(needs_layout_passes=False),
)(_sc_gather_body)


def _tc_loss_body(hm_p_ref, hm_g_ref,
                  gwh_ref, whg_ref, mwh_ref,
                  greg_ref, regg_ref, mreg_ref,
                  gct_ref, ctg_ref, mct_ref,
                  out_ref):
    i = pl.program_id(0)

    @pl.when(i == 0)
    def _init():
        def smooth_l1_sum(p, t, m):
            d = jnp.abs(p - t)
            l = jnp.where(d < 1.0, 0.5 * d * d, d - 0.5)
            return jnp.sum(l * m)

        out_ref[0] = 0.0
        out_ref[1] = 0.0
        out_ref[2] = 0.0
        out_ref[3] = smooth_l1_sum(gwh_ref[...], whg_ref[...], mwh_ref[...])
        out_ref[4] = smooth_l1_sum(greg_ref[...], regg_ref[...], mreg_ref[...])
        pc = jnp.clip(gct_ref[...], _EPS, 1.0 - _EPS)
        t = ctg_ref[...]
        bce = -(t * jnp.log(pc) + (1.0 - t) * jnp.log(1.0 - pc))
        out_ref[5] = jnp.sum(bce * mct_ref[...])
        out_ref[6] = jnp.sum(mct_ref[...])
        out_ref[7] = 0.0

    p = hm_p_ref[...]
    g = hm_g_ref[...]
    pos = g == 1.0
    omp = 1.0 - p
    l = jnp.log(jnp.where(pos, p, omp))
    ng = 1.0 - g
    ng2 = ng * ng
    w = jnp.where(pos, omp * omp, p * p * (ng2 * ng2))
    lw = l * w
    posf = pos.astype(jnp.float32)
    s_pos = jnp.sum(lw * posf)
    s_all = jnp.sum(lw)
    n_pos = jnp.sum(posf)
    out_ref[0] += s_pos
    out_ref[1] += s_all - s_pos
    out_ref[2] += n_pos


def _tc_loss(hm_p2, hm_g2, gwh, whg, mwh, greg, regg, mreg, gct, ctg, mct):
    rows = hm_p2.shape[0]
    br = 24
    grid = (rows // br,)
    hm_spec = pl.BlockSpec((br, _HW), lambda i: (i, 0))

    def full(a):
        return pl.BlockSpec(a.shape, lambda i: (0, 0))

    return pl.pallas_call(
        _tc_loss_body,
        grid=grid,
        in_specs=[hm_spec, hm_spec,
                  full(gwh), full(whg), full(mwh),
                  full(greg), full(regg), full(mreg),
                  full(gct), full(ctg), full(mct)],
        out_specs=pl.BlockSpec(memory_space=pltpu.SMEM),
        out_shape=jax.ShapeDtypeStruct((8,), jnp.float32),
        compiler_params=pltpu.CompilerParams(
            dimension_semantics=("arbitrary",)),
    )(hm_p2, hm_g2, gwh, whg, mwh, greg, regg, mreg, gct, ctg, mct)


def kernel(hm_pred, hm_gt, wh_pred, wh_gt, reg_pred, reg_gt,
           cls_theta_pred, cls_theta_gt, reg_mask, ind):
    B, K = ind.shape
    pad = _KP - K

    wh2 = wh_pred.reshape(B * _CWH, _HW)
    reg2 = reg_pred.reshape(B * _CREG, _HW)
    ct2 = cls_theta_pred.reshape(B, _HW)
    hm_p2 = hm_pred.reshape(-1, _HW)
    hm_g2 = hm_gt.reshape(-1, _HW)

    idx_p = jnp.pad(ind, ((0, 0), (0, pad)))
    mask_p = jnp.pad(reg_mask, ((0, 0), (0, pad)))

    whg = jnp.pad(wh_gt, ((0, 0), (0, pad), (0, 0))).transpose(0, 2, 1)
    whg = whg.reshape(B * _CWH, _KP)
    regg = jnp.pad(reg_gt, ((0, 0), (0, pad), (0, 0))).transpose(0, 2, 1)
    regg = regg.reshape(B * _CREG, _KP)
    ctg = jnp.pad(cls_theta_gt, ((0, 0), (0, pad), (0, 0))).transpose(0, 2, 1)
    ctg = ctg.reshape(B, _KP)
    mwh = jnp.repeat(mask_p, _CWH, axis=0)
    mreg = jnp.repeat(mask_p, _CREG, axis=0)

    g_wh, g_reg, g_ct = _sc_gather(wh2, reg2, ct2, idx_p)

    s = _tc_loss(hm_p2, hm_g2, g_wh, whg, mwh, g_reg, regg, mreg,
                 g_ct, ctg, mask_p)

    pos_s, neg_s, n_pos = s[0], s[1], s[2]
    wh_s, reg_s, ct_s, m_s = s[3], s[4], s[5], s[6]
    hm_loss = jnp.where(n_pos == 0, -neg_s,
                        -(pos_s + neg_s) / jnp.maximum(n_pos, 1.0))
    wh_loss = wh_s / jnp.maximum(m_s * _CWH, 1.0)
    off_loss = reg_s / jnp.maximum(m_s * _CREG, 1.0)
    ct_loss = ct_s / jnp.maximum(m_s, 1.0)
    total = hm_loss + wh_loss + off_loss + ct_loss
    return jnp.stack([hm_loss, off_loss, wh_loss, ct_loss, total])
